# baseline (speedup 1.0000x reference)
import functools

import numpy as np
import jax
import jax.numpy as jnp
from jax import lax
from jax.experimental import pallas as pl
from jax.experimental.pallas import tpu as pltpu
from jax.experimental.pallas import tpu_sc as plsc

_z = np.linspace(-12.0, 12.0, 200001)
_pdf = np.exp(-0.5 * _z ** 2) / np.sqrt(2.0 * np.pi)
_silu_np = _z / (1.0 + np.exp(-_z))
_ACT_CST = float(1.0 / np.sqrt(np.trapz(_silu_np ** 2 * _pdf, _z)))

_NC, _NS = 2, 16
_NW = _NC * _NS


def _act(x):
    return jax.nn.silu(x) * _ACT_CST


def _bdot(a, b):
    return jnp.dot(a.astype(jnp.bfloat16), b.astype(jnp.bfloat16),
                   preferred_element_type=jnp.float32)


def _mlp_body(eet_ref, eft_ref, w0_ref, w1_ref, w2_ref, r_ref, t_ref):
    BE, D = t_ref.shape
    h = _act(lax.dot_general(
        eet_ref[...].astype(jnp.bfloat16), w0_ref[...].astype(jnp.bfloat16),
        (((0,), (0,)), ((), ())), preferred_element_type=jnp.float32))
    h = _act(_bdot(h, w1_ref[...]))
    s = _bdot(h, w2_ref[...])
    efb = lax.dot_general(eft_ref[...].astype(jnp.bfloat16),
                          r_ref[...].astype(jnp.bfloat16),
                          (((0,), (0,)), ((), ())),
                          preferred_element_type=jnp.float32)
    t = s[:, 0:D] * efb[:, 0:D]
    for v in range(1, 4):
        t += s[:, v * D:(v + 1) * D] * efb[:, v * D:(v + 1) * D]
    t_ref[...] = t


def _mlp_t(eet, eft, w0n, w1n, w2g, block_e=3200):
    RE, E = eet.shape
    DE = eft.shape[0]
    D = w2g.shape[1] // DE
    grid = E // block_e
    r = np.zeros((DE, DE * D), np.float32)
    for v in range(DE):
        r[v, v * D:(v + 1) * D] = 1.0
    r = jnp.asarray(r)
    return pl.pallas_call(
        _mlp_body,
        grid=(grid,),
        in_specs=[
            pl.BlockSpec((RE, block_e), lambda i: (0, i)),
            pl.BlockSpec((DE, block_e), lambda i: (0, i)),
            pl.BlockSpec((RE, w0n.shape[1]), lambda i: (0, 0)),
            pl.BlockSpec(w1n.shape, lambda i: (0, 0)),
            pl.BlockSpec(w2g.shape, lambda i: (0, 0)),
            pl.BlockSpec(r.shape, lambda i: (0, 0)),
        ],
        out_specs=pl.BlockSpec((block_e, D), lambda i: (i, 0)),
        out_shape=jax.ShapeDtypeStruct((E, D), jnp.float32),
    )(eet, eft, w0n, w1n, w2g, r)


def _sc_gcn(src, dst, t, node_feat, ch=64):
    N, D = node_feat.shape
    E = src.shape[0]
    nch = E // ch
    assert E % ch == 0 and ch % 8 == 0
    cpt_max = -(-nch // _NW)
    stripe = 632
    last_stripe = N - (_NS - 1) * stripe
    assert stripe % 8 == 0 and last_stripe % 8 == 0 and 0 < last_stripe
    mesh = plsc.VectorSubcoreMesh(core_axis_name="c", subcore_axis_name="s",
                                  num_cores=_NC, num_subcores=_NS)

    @functools.partial(
        pl.kernel,
        mesh=mesh,
        out_type=jax.ShapeDtypeStruct((_NC, N, D), jnp.float32),
        scratch_types=(
            [pltpu.VMEM((ch,), jnp.int32)] * 3
            + [pltpu.VMEM((ch,), jnp.int32)] * 3
            + [pltpu.VMEM((ch, D), jnp.float32)] * 3
            + [pltpu.VMEM((ch, D), jnp.float32)] * 3
            + [pltpu.VMEM_SHARED((N, D), jnp.float32)]
            + [pltpu.SemaphoreType.DMA] * 15
        ),
    )
    def k(src_hbm, dst_hbm, t_hbm, nf_hbm, out_hbm,
          si0, si1, si2, di0, di1, di2, xg0, xg1, xg2, tv0, tv1, tv2, acc,
          ssi0, ssi1, ssi2, sdi0, sdi1, sdi2, sg0, sg1, sg2,
          st0, st1, st2, ss0, ss1, ss2):
        cid = lax.axis_index("c")
        sid = lax.axis_index("s")
        wid = sid * _NC + cid
        si = (si0, si1, si2)
        di = (di0, di1, di2)
        xg = (xg0, xg1, xg2)
        tv = (tv0, tv1, tv2)
        ssi = (ssi0, ssi1, ssi2)
        sdi = (sdi0, sdi1, sdi2)
        sg = (sg0, sg1, sg2)
        st = (st0, st1, st2)
        ss = (ss0, ss1, ss2)

        zero = jnp.zeros((16,), jnp.float32)

        def zrow(i, _):
            r = i // (D // 16)
            j = i % (D // 16)
            xg0[r, pl.ds(j * 16, 16)] = zero
            return 0
        lax.fori_loop(0, ch * (D // 16), zrow, 0)

        r0 = sid * stripe

        def zfill(total):
            full, rem = total // ch, total % ch
            for q in range(full):
                pltpu.sync_copy(xg0, acc.at[pl.ds(r0 + q * ch, ch)])
            if rem:
                pltpu.sync_copy(xg0.at[pl.ds(0, rem)],
                                acc.at[pl.ds(r0 + full * ch, rem)])

        @pl.when(sid < _NS - 1)
        def _():
            zfill(stripe)

        @pl.when(sid == _NS - 1)
        def _():
            zfill(last_stripe)

        plsc.subcore_barrier()

        n_me = (nch - wid + _NW - 1) // _NW

        def e_of(g):
            return (wid + g * _NW) * ch

        def start_idx(g, b):
            pltpu.async_copy(src_hbm.at[pl.ds(e_of(g), ch)], si[b], ssi[b])
            pltpu.async_copy(dst_hbm.at[pl.ds(e_of(g), ch)], di[b], sdi[b])

        def wait_idx(g, b):
            pltpu.make_async_copy(src_hbm.at[pl.ds(e_of(g), ch)], si[b], ssi[b]).wait()
            pltpu.make_async_copy(dst_hbm.at[pl.ds(e_of(g), ch)], di[b], sdi[b]).wait()

        def start_gt(g, b):
            pltpu.async_copy(nf_hbm.at[si[b]], xg[b], sg[b])
            pltpu.async_copy(t_hbm.at[pl.ds(e_of(g), ch)], tv[b], st[b])

        def wait_gt(g, b):
            pltpu.make_async_copy(nf_hbm.at[si[b]], xg[b], sg[b]).wait()
            pltpu.make_async_copy(t_hbm.at[pl.ds(e_of(g), ch)], tv[b], st[b]).wait()

        def start_sc(b):
            pltpu.async_copy(xg[b], acc.at[di[b]], ss[b], add=True)

        def wait_sc(b):
            pltpu.make_async_copy(xg[b], acc.at[di[b]], ss[b]).wait()

        start_idx(0, 0)
        wait_idx(0, 0)
        start_gt(0, 0)

        @pl.when(1 < n_me)
        def _():
            start_idx(1, 1)

        def outer(go, _):
            for b in range(3):
                g = go * 3 + b

                @pl.when(g < n_me)
                def _():
                    wait_gt(g, b)

                    @pl.when(g + 1 < n_me)
                    def _():
                        wait_idx(g + 1, (b + 1) % 3)
                        start_gt(g + 1, (b + 1) % 3)

                    def erow(e, _):
                        for k in range(2):
                            for j in range(D // 16):
                                sl = pl.ds(j * 16, 16)
                                xg[b][2 * e + k, sl] = (xg[b][2 * e + k, sl]
                                                        * tv[b][2 * e + k, sl])
                        return 0
                    lax.fori_loop(0, ch // 2, erow, 0)
                    start_sc(b)

                    @pl.when(g + 2 < n_me)
                    def _():
                        @pl.when(g >= 1)
                        def _():
                            wait_sc((b + 2) % 3)
                        start_idx(g + 2, (b + 2) % 3)
            return 0
        lax.fori_loop(0, (cpt_max + 2) // 3, outer, 0)

        for b in range(3):
            @pl.when(n_me > b)
            def _(b=b):
                wait_sc(b)

        plsc.subcore_barrier()

        @pl.when(sid < _NS - 1)
        def _():
            pltpu.sync_copy(acc.at[pl.ds(r0, stripe)],
                            out_hbm.at[cid, pl.ds(r0, stripe)])

        @pl.when(sid == _NS - 1)
        def _():
            pltpu.sync_copy(acc.at[pl.ds(r0, last_stripe)],
                            out_hbm.at[cid, pl.ds(r0, last_stripe)])

    return k(src, dst, t, node_feat)


def _combine_body(*refs):
    o_ref = refs[-1]
    nf_ref, w_ref = refs[-3], refs[-2]
    acc = jnp.dot(nf_ref[...], w_ref[...], preferred_element_type=jnp.float32)
    for p_ref in refs[:-3]:
        acc += p_ref[0] + p_ref[1]
    o_ref[...] = acc


def _combine(parts, node_feat, scn, block_n=2000):
    N, D = node_feat.shape
    grid = N // block_n
    return pl.pallas_call(
        _combine_body,
        grid=(grid,),
        in_specs=(
            [pl.BlockSpec((_NC, block_n, D), lambda i: (0, i, 0))] * len(parts)
            + [pl.BlockSpec((block_n, D), lambda i: (i, 0)),
               pl.BlockSpec((D, D), lambda i: (0, 0))]
        ),
        out_specs=pl.BlockSpec((block_n, D), lambda i: (i, 0)),
        out_shape=jax.ShapeDtypeStruct((N, D), jnp.float32),
    )(*parts, node_feat, scn)


def kernel(edge_index, node_feat, edge_feat, edge_embed, dim_size, fc_w0, fc_w1, fc_w2, sc_w):
    N, D = node_feat.shape
    E, DE = edge_feat.shape
    RE = edge_embed.shape[1]
    H = fc_w0.shape[1]

    w0n = fc_w0 * (1.0 / np.sqrt(RE))
    w1n = fc_w1 * (1.0 / np.sqrt(H))
    alpha = 1.0 / np.sqrt(DE)
    w2g = (fc_w2 * (alpha / np.sqrt(H))).reshape(H, D, DE).transpose(0, 2, 1).reshape(H, DE * D)
    scn = sc_w * (1.0 / np.sqrt(D))

    eet, eft = edge_embed.T, edge_feat.T
    src, dst = edge_index[0], edge_index[1]
    nsplit = 4
    h = E // nsplit
    parts = []
    for k in range(nsplit):
        sl = slice(k * h, (k + 1) * h)
        tk = _mlp_t(eet[:, sl], eft[:, sl], w0n, w1n, w2g)
        parts.append(_sc_gcn(src[sl], dst[sl], tk, node_feat))

    return _combine(parts, node_feat, scn)


# speedup vs baseline: 1.0330x; 1.0330x over previous
import functools

import numpy as np
import jax
import jax.numpy as jnp
from jax import lax
from jax.experimental import pallas as pl
from jax.experimental.pallas import tpu as pltpu
from jax.experimental.pallas import tpu_sc as plsc

_z = np.linspace(-12.0, 12.0, 200001)
_pdf = np.exp(-0.5 * _z ** 2) / np.sqrt(2.0 * np.pi)
_silu_np = _z / (1.0 + np.exp(-_z))
_ACT_CST = float(1.0 / np.sqrt(np.trapz(_silu_np ** 2 * _pdf, _z)))

_NC, _NS = 2, 16
_NW = _NC * _NS


def _act(x):
    return jax.nn.silu(x) * _ACT_CST


def _bdot(a, b):
    return jnp.dot(a.astype(jnp.bfloat16), b.astype(jnp.bfloat16),
                   preferred_element_type=jnp.float32)


def _mlp_body(eet_ref, eft_ref, w0_ref, w1_ref, w2_ref, r_ref, t_ref):
    BE, D = t_ref.shape
    h = _act(lax.dot_general(
        eet_ref[...].astype(jnp.bfloat16), w0_ref[...].astype(jnp.bfloat16),
        (((0,), (0,)), ((), ())), preferred_element_type=jnp.float32))
    h = _act(_bdot(h, w1_ref[...]))
    s = _bdot(h, w2_ref[...])
    efb = lax.dot_general(eft_ref[...].astype(jnp.bfloat16),
                          r_ref[...].astype(jnp.bfloat16),
                          (((0,), (0,)), ((), ())),
                          preferred_element_type=jnp.float32)
    t = s[:, 0:D] * efb[:, 0:D]
    for v in range(1, 4):
        t += s[:, v * D:(v + 1) * D] * efb[:, v * D:(v + 1) * D]
    t_ref[...] = t


def _mlp_t(eet, eft, w0n, w1n, w2g, block_e=3200):
    RE, E = eet.shape
    DE = eft.shape[0]
    D = w2g.shape[1] // DE
    grid = E // block_e
    r = np.zeros((DE, DE * D), np.float32)
    for v in range(DE):
        r[v, v * D:(v + 1) * D] = 1.0
    r = jnp.asarray(r)
    return pl.pallas_call(
        _mlp_body,
        grid=(grid,),
        in_specs=[
            pl.BlockSpec((RE, block_e), lambda i: (0, i)),
            pl.BlockSpec((DE, block_e), lambda i: (0, i)),
            pl.BlockSpec((RE, w0n.shape[1]), lambda i: (0, 0)),
            pl.BlockSpec(w1n.shape, lambda i: (0, 0)),
            pl.BlockSpec(w2g.shape, lambda i: (0, 0)),
            pl.BlockSpec(r.shape, lambda i: (0, 0)),
        ],
        out_specs=pl.BlockSpec((block_e, D), lambda i: (i, 0)),
        out_shape=jax.ShapeDtypeStruct((E, D), jnp.float32),
    )(eet, eft, w0n, w1n, w2g, r)


def _sc_gcn(src, dst, t, node_feat, ch=64):
    N, D = node_feat.shape
    E = src.shape[0]
    nch = E // ch
    assert E % ch == 0 and ch % 8 == 0
    cpt_max = -(-nch // _NW)
    stripe = 632
    last_stripe = N - (_NS - 1) * stripe
    assert stripe % 8 == 0 and last_stripe % 8 == 0 and 0 < last_stripe
    mesh = plsc.VectorSubcoreMesh(core_axis_name="c", subcore_axis_name="s",
                                  num_cores=_NC, num_subcores=_NS)

    @functools.partial(
        pl.kernel,
        mesh=mesh,
        out_type=jax.ShapeDtypeStruct((_NC, N, D), jnp.float32),
        scratch_types=(
            [pltpu.VMEM((ch,), jnp.int32)] * 3
            + [pltpu.VMEM((ch,), jnp.int32)] * 3
            + [pltpu.VMEM((ch, D), jnp.float32)] * 3
            + [pltpu.VMEM((ch, D), jnp.float32)] * 3
            + [pltpu.VMEM_SHARED((N, D), jnp.float32)]
            + [pltpu.SemaphoreType.DMA] * 15
        ),
    )
    def k(src_hbm, dst_hbm, t_hbm, nf_hbm, out_hbm,
          si0, si1, si2, di0, di1, di2, xg0, xg1, xg2, tv0, tv1, tv2, acc,
          ssi0, ssi1, ssi2, sdi0, sdi1, sdi2, sg0, sg1, sg2,
          st0, st1, st2, ss0, ss1, ss2):
        cid = lax.axis_index("c")
        sid = lax.axis_index("s")
        wid = sid * _NC + cid
        si = (si0, si1, si2)
        di = (di0, di1, di2)
        xg = (xg0, xg1, xg2)
        tv = (tv0, tv1, tv2)
        ssi = (ssi0, ssi1, ssi2)
        sdi = (sdi0, sdi1, sdi2)
        sg = (sg0, sg1, sg2)
        st = (st0, st1, st2)
        ss = (ss0, ss1, ss2)

        zero = jnp.zeros((16,), jnp.float32)

        def zrow(i, _):
            r = i // (D // 16)
            j = i % (D // 16)
            xg0[r, pl.ds(j * 16, 16)] = zero
            return 0
        lax.fori_loop(0, ch * (D // 16), zrow, 0)

        r0 = sid * stripe

        def zfill(total):
            full, rem = total // ch, total % ch
            for q in range(full):
                pltpu.sync_copy(xg0, acc.at[pl.ds(r0 + q * ch, ch)])
            if rem:
                pltpu.sync_copy(xg0.at[pl.ds(0, rem)],
                                acc.at[pl.ds(r0 + full * ch, rem)])

        @pl.when(sid < _NS - 1)
        def _():
            zfill(stripe)

        @pl.when(sid == _NS - 1)
        def _():
            zfill(last_stripe)

        plsc.subcore_barrier()

        n_me = (nch - wid + _NW - 1) // _NW

        def e_of(g):
            return (wid + g * _NW) * ch

        def start_idx(g, b):
            pltpu.async_copy(src_hbm.at[pl.ds(e_of(g), ch)], si[b], ssi[b])
            pltpu.async_copy(dst_hbm.at[pl.ds(e_of(g), ch)], di[b], sdi[b])

        def wait_idx(g, b):
            pltpu.make_async_copy(src_hbm.at[pl.ds(e_of(g), ch)], si[b], ssi[b]).wait()
            pltpu.make_async_copy(dst_hbm.at[pl.ds(e_of(g), ch)], di[b], sdi[b]).wait()

        def start_gt(g, b):
            pltpu.async_copy(nf_hbm.at[si[b]], xg[b], sg[b])
            pltpu.async_copy(t_hbm.at[pl.ds(e_of(g), ch)], tv[b], st[b])

        def wait_gt(g, b):
            pltpu.make_async_copy(nf_hbm.at[si[b]], xg[b], sg[b]).wait()
            pltpu.make_async_copy(t_hbm.at[pl.ds(e_of(g), ch)], tv[b], st[b]).wait()

        def start_sc(b):
            pltpu.async_copy(xg[b], acc.at[di[b]], ss[b], add=True)

        def wait_sc(b):
            pltpu.make_async_copy(xg[b], acc.at[di[b]], ss[b]).wait()

        start_idx(0, 0)
        wait_idx(0, 0)
        start_gt(0, 0)

        @pl.when(1 < n_me)
        def _():
            start_idx(1, 1)

        def outer(go, _):
            for b in range(3):
                g = go * 3 + b

                @pl.when(g < n_me)
                def _():
                    wait_gt(g, b)

                    @pl.when(g + 1 < n_me)
                    def _():
                        wait_idx(g + 1, (b + 1) % 3)
                        start_gt(g + 1, (b + 1) % 3)

                    def erow(e, _):
                        for k in range(2):
                            for j in range(D // 16):
                                sl = pl.ds(j * 16, 16)
                                xg[b][2 * e + k, sl] = (xg[b][2 * e + k, sl]
                                                        * tv[b][2 * e + k, sl])
                        return 0
                    lax.fori_loop(0, ch // 2, erow, 0)
                    start_sc(b)

                    @pl.when(g + 2 < n_me)
                    def _():
                        @pl.when(g >= 1)
                        def _():
                            wait_sc((b + 2) % 3)
                        start_idx(g + 2, (b + 2) % 3)
            return 0
        lax.fori_loop(0, (cpt_max + 2) // 3, outer, 0)

        for b in range(3):
            @pl.when(n_me > b)
            def _(b=b):
                wait_sc(b)

        plsc.subcore_barrier()

        @pl.when(sid < _NS - 1)
        def _():
            pltpu.sync_copy(acc.at[pl.ds(r0, stripe)],
                            out_hbm.at[cid, pl.ds(r0, stripe)])

        @pl.when(sid == _NS - 1)
        def _():
            pltpu.sync_copy(acc.at[pl.ds(r0, last_stripe)],
                            out_hbm.at[cid, pl.ds(r0, last_stripe)])

    return k(src, dst, t, node_feat)


def _combine_body(*refs):
    o_ref = refs[-1]
    nf_ref, w_ref = refs[-3], refs[-2]
    acc = jnp.dot(nf_ref[...], w_ref[...], preferred_element_type=jnp.float32)
    for p_ref in refs[:-3]:
        acc += p_ref[0] + p_ref[1]
    o_ref[...] = acc


def _combine(parts, node_feat, scn, block_n=2000):
    N, D = node_feat.shape
    grid = N // block_n
    return pl.pallas_call(
        _combine_body,
        grid=(grid,),
        in_specs=(
            [pl.BlockSpec((_NC, block_n, D), lambda i: (0, i, 0))] * len(parts)
            + [pl.BlockSpec((block_n, D), lambda i: (i, 0)),
               pl.BlockSpec((D, D), lambda i: (0, 0))]
        ),
        out_specs=pl.BlockSpec((block_n, D), lambda i: (i, 0)),
        out_shape=jax.ShapeDtypeStruct((N, D), jnp.float32),
    )(*parts, node_feat, scn)


def kernel(edge_index, node_feat, edge_feat, edge_embed, dim_size, fc_w0, fc_w1, fc_w2, sc_w):
    N, D = node_feat.shape
    E, DE = edge_feat.shape
    RE = edge_embed.shape[1]
    H = fc_w0.shape[1]

    w0n = fc_w0 * (1.0 / np.sqrt(RE))
    w1n = fc_w1 * (1.0 / np.sqrt(H))
    alpha = 1.0 / np.sqrt(DE)
    w2g = (fc_w2 * (alpha / np.sqrt(H))).reshape(H, D, DE).transpose(0, 2, 1).reshape(H, DE * D)
    scn = sc_w * (1.0 / np.sqrt(D))

    eet, eft = edge_embed.T, edge_feat.T
    src, dst = edge_index[0], edge_index[1]
    nsplit = 2
    h = E // nsplit
    assert h % 3200 == 0 and h % 64 == 0
    parts = []
    for k in range(nsplit):
        sl = slice(k * h, (k + 1) * h)
        tk = _mlp_t(eet[:, sl], eft[:, sl], w0n, w1n, w2g)
        parts.append(_sc_gcn(src[sl], dst[sl], tk, node_feat))

    return _combine(parts, node_feat, scn)


# speedup vs baseline: 1.0490x; 1.0154x over previous
import functools

import numpy as np
import jax
import jax.numpy as jnp
from jax import lax
from jax.experimental import pallas as pl
from jax.experimental.pallas import tpu as pltpu
from jax.experimental.pallas import tpu_sc as plsc

_z = np.linspace(-12.0, 12.0, 200001)
_pdf = np.exp(-0.5 * _z ** 2) / np.sqrt(2.0 * np.pi)
_silu_np = _z / (1.0 + np.exp(-_z))
_ACT_CST = float(1.0 / np.sqrt(np.trapz(_silu_np ** 2 * _pdf, _z)))

_NC, _NS = 2, 16
_NW = _NC * _NS


def _act(x):
    return jax.nn.silu(x) * _ACT_CST


def _bdot(a, b):
    return jnp.dot(a.astype(jnp.bfloat16), b.astype(jnp.bfloat16),
                   preferred_element_type=jnp.float32)


def _mlp_body(eet_ref, eft_ref, w0_ref, w1_ref, w2_ref, r_ref, t_ref):
    BE, D = t_ref.shape
    h = _act(lax.dot_general(
        eet_ref[...].astype(jnp.bfloat16), w0_ref[...].astype(jnp.bfloat16),
        (((0,), (0,)), ((), ())), preferred_element_type=jnp.float32))
    h = _act(_bdot(h, w1_ref[...]))
    s = _bdot(h, w2_ref[...])
    efb = lax.dot_general(eft_ref[...].astype(jnp.bfloat16),
                          r_ref[...].astype(jnp.bfloat16),
                          (((0,), (0,)), ((), ())),
                          preferred_element_type=jnp.float32)
    t = s[:, 0:D] * efb[:, 0:D]
    for v in range(1, 4):
        t += s[:, v * D:(v + 1) * D] * efb[:, v * D:(v + 1) * D]
    t_ref[...] = t


def _mlp_t(eet, eft, w0n, w1n, w2g, block_e=3200):
    RE, E = eet.shape
    DE = eft.shape[0]
    D = w2g.shape[1] // DE
    grid = E // block_e
    r = np.zeros((DE, DE * D), np.float32)
    for v in range(DE):
        r[v, v * D:(v + 1) * D] = 1.0
    r = jnp.asarray(r)
    return pl.pallas_call(
        _mlp_body,
        grid=(grid,),
        in_specs=[
            pl.BlockSpec((RE, block_e), lambda i: (0, i)),
            pl.BlockSpec((DE, block_e), lambda i: (0, i)),
            pl.BlockSpec((RE, w0n.shape[1]), lambda i: (0, 0)),
            pl.BlockSpec(w1n.shape, lambda i: (0, 0)),
            pl.BlockSpec(w2g.shape, lambda i: (0, 0)),
            pl.BlockSpec(r.shape, lambda i: (0, 0)),
        ],
        out_specs=pl.BlockSpec((block_e, D), lambda i: (i, 0)),
        out_shape=jax.ShapeDtypeStruct((E, D), jnp.float32),
    )(eet, eft, w0n, w1n, w2g, r)


def _sc_gcn(src, dst, t, node_feat, ch=64):
    N, D = node_feat.shape
    E = src.shape[0]
    nch = E // ch
    assert E % ch == 0 and ch % 8 == 0
    cpt_max = -(-nch // _NW)
    stripe = 632
    last_stripe = N - (_NS - 1) * stripe
    assert stripe % 8 == 0 and last_stripe % 8 == 0 and 0 < last_stripe
    mesh = plsc.VectorSubcoreMesh(core_axis_name="c", subcore_axis_name="s",
                                  num_cores=_NC, num_subcores=_NS)

    @functools.partial(
        pl.kernel,
        mesh=mesh,
        out_type=jax.ShapeDtypeStruct((_NC, N, D), jnp.float32),
        scratch_types=(
            [pltpu.VMEM((ch,), jnp.int32)] * 3
            + [pltpu.VMEM((ch,), jnp.int32)] * 3
            + [pltpu.VMEM((ch, D), jnp.float32)] * 3
            + [pltpu.VMEM((ch, D), jnp.float32)] * 3
            + [pltpu.VMEM_SHARED((N, D), jnp.float32)]
            + [pltpu.SemaphoreType.DMA] * 15
        ),
    )
    def k(src_hbm, dst_hbm, t_hbm, nf_hbm, out_hbm,
          si0, si1, si2, di0, di1, di2, xg0, xg1, xg2, tv0, tv1, tv2, acc,
          ssi0, ssi1, ssi2, sdi0, sdi1, sdi2, sg0, sg1, sg2,
          st0, st1, st2, ss0, ss1, ss2):
        cid = lax.axis_index("c")
        sid = lax.axis_index("s")
        wid = sid * _NC + cid
        si = (si0, si1, si2)
        di = (di0, di1, di2)
        xg = (xg0, xg1, xg2)
        tv = (tv0, tv1, tv2)
        ssi = (ssi0, ssi1, ssi2)
        sdi = (sdi0, sdi1, sdi2)
        sg = (sg0, sg1, sg2)
        st = (st0, st1, st2)
        ss = (ss0, ss1, ss2)

        zero = jnp.zeros((16,), jnp.float32)

        def zrow(i, _):
            r = i // (D // 16)
            j = i % (D // 16)
            xg0[r, pl.ds(j * 16, 16)] = zero
            return 0
        lax.fori_loop(0, ch * (D // 16), zrow, 0)

        r0 = sid * stripe

        def zfill(total):
            full, rem = total // ch, total % ch
            for q in range(full):
                pltpu.sync_copy(xg0, acc.at[pl.ds(r0 + q * ch, ch)])
            if rem:
                pltpu.sync_copy(xg0.at[pl.ds(0, rem)],
                                acc.at[pl.ds(r0 + full * ch, rem)])

        @pl.when(sid < _NS - 1)
        def _():
            zfill(stripe)

        @pl.when(sid == _NS - 1)
        def _():
            zfill(last_stripe)

        plsc.subcore_barrier()

        n_me = (nch - wid + _NW - 1) // _NW

        def e_of(g):
            return (wid + g * _NW) * ch

        def start_idx(g, b):
            pltpu.async_copy(src_hbm.at[pl.ds(e_of(g), ch)], si[b], ssi[b])
            pltpu.async_copy(dst_hbm.at[pl.ds(e_of(g), ch)], di[b], sdi[b])

        def wait_idx(g, b):
            pltpu.make_async_copy(src_hbm.at[pl.ds(e_of(g), ch)], si[b], ssi[b]).wait()
            pltpu.make_async_copy(dst_hbm.at[pl.ds(e_of(g), ch)], di[b], sdi[b]).wait()

        def start_gt(g, b):
            pltpu.async_copy(nf_hbm.at[si[b]], xg[b], sg[b])
            pltpu.async_copy(t_hbm.at[pl.ds(e_of(g), ch)], tv[b], st[b])

        def wait_gt(g, b):
            pltpu.make_async_copy(nf_hbm.at[si[b]], xg[b], sg[b]).wait()
            pltpu.make_async_copy(t_hbm.at[pl.ds(e_of(g), ch)], tv[b], st[b]).wait()

        def start_sc(b):
            pltpu.async_copy(xg[b], acc.at[di[b]], ss[b], add=True)

        def wait_sc(b):
            pltpu.make_async_copy(xg[b], acc.at[di[b]], ss[b]).wait()

        start_idx(0, 0)
        wait_idx(0, 0)
        start_gt(0, 0)

        @pl.when(1 < n_me)
        def _():
            start_idx(1, 1)

        def outer(go, _):
            for b in range(3):
                g = go * 3 + b

                @pl.when(g < n_me)
                def _():
                    wait_gt(g, b)

                    @pl.when(g + 1 < n_me)
                    def _():
                        wait_idx(g + 1, (b + 1) % 3)
                        start_gt(g + 1, (b + 1) % 3)

                    def erow(e, _):
                        for k in range(2):
                            for j in range(D // 16):
                                sl = pl.ds(j * 16, 16)
                                xg[b][2 * e + k, sl] = (xg[b][2 * e + k, sl]
                                                        * tv[b][2 * e + k, sl])
                        return 0
                    lax.fori_loop(0, ch // 2, erow, 0)
                    start_sc(b)

                    @pl.when(g + 2 < n_me)
                    def _():
                        @pl.when(g >= 1)
                        def _():
                            wait_sc((b + 2) % 3)
                        start_idx(g + 2, (b + 2) % 3)
            return 0
        lax.fori_loop(0, (cpt_max + 2) // 3, outer, 0)

        for b in range(3):
            @pl.when(n_me > b)
            def _(b=b):
                wait_sc(b)

        plsc.subcore_barrier()

        @pl.when(sid < _NS - 1)
        def _():
            pltpu.sync_copy(acc.at[pl.ds(r0, stripe)],
                            out_hbm.at[cid, pl.ds(r0, stripe)])

        @pl.when(sid == _NS - 1)
        def _():
            pltpu.sync_copy(acc.at[pl.ds(r0, last_stripe)],
                            out_hbm.at[cid, pl.ds(r0, last_stripe)])

    return k(src, dst, t, node_feat)


def _combine_body(*refs):
    o_ref = refs[-1]
    nf_ref, w_ref = refs[-3], refs[-2]
    acc = jnp.dot(nf_ref[...], w_ref[...], preferred_element_type=jnp.float32)
    for p_ref in refs[:-3]:
        acc += p_ref[0] + p_ref[1]
    o_ref[...] = acc


def _combine(parts, node_feat, scn, block_n=2000):
    N, D = node_feat.shape
    grid = N // block_n
    return pl.pallas_call(
        _combine_body,
        grid=(grid,),
        in_specs=(
            [pl.BlockSpec((_NC, block_n, D), lambda i: (0, i, 0))] * len(parts)
            + [pl.BlockSpec((block_n, D), lambda i: (i, 0)),
               pl.BlockSpec((D, D), lambda i: (0, 0))]
        ),
        out_specs=pl.BlockSpec((block_n, D), lambda i: (i, 0)),
        out_shape=jax.ShapeDtypeStruct((N, D), jnp.float32),
    )(*parts, node_feat, scn)


def kernel(edge_index, node_feat, edge_feat, edge_embed, dim_size, fc_w0, fc_w1, fc_w2, sc_w):
    N, D = node_feat.shape
    E, DE = edge_feat.shape
    RE = edge_embed.shape[1]
    H = fc_w0.shape[1]

    w0n = fc_w0 * (1.0 / np.sqrt(RE))
    w1n = fc_w1 * (1.0 / np.sqrt(H))
    alpha = 1.0 / np.sqrt(DE)
    w2g = (fc_w2 * (alpha / np.sqrt(H))).reshape(H, D, DE).transpose(0, 2, 1).reshape(H, DE * D)
    scn = sc_w * (1.0 / np.sqrt(D))

    eet, eft = edge_embed.T, edge_feat.T
    src, dst = edge_index[0], edge_index[1]
    sizes = [57600, 51200, 51200] if E == 160000 else [E // 2, E - E // 2]
    parts = []
    off = 0
    for sz in sizes:
        assert sz % 3200 == 0 and sz % 64 == 0
        sl = slice(off, off + sz)
        off += sz
        tk = _mlp_t(eet[:, sl], eft[:, sl], w0n, w1n, w2g)
        parts.append(_sc_gcn(src[sl], dst[sl], tk, node_feat))

    return _combine(parts, node_feat, scn)
